# Initial kernel scaffold; baseline (speedup 1.0000x reference)
import jax
import jax.numpy as jnp
from jax import lax
from jax.experimental import pallas as pl
from jax.experimental.pallas import tpu as pltpu
from jax.experimental.pallas import tpu_sc as plsc

N = 10000
E = 320000
D = 128

NC = 2
NS = 16
NW = NC * NS
EPW = E // NW
CHUNK = 80
NCHUNK = EPW // CHUNK
ROWS_PER_TILE = N // NS


def _sc_body(node_hbm, edge_hbm, src_hbm, dst_hbm, zeros_hbm, out_hbm,
             idx_src_v, idx_dst_v, node_rows_v, edge_rows_v, acc_spmem, sem):
    c = lax.axis_index("c")
    s = lax.axis_index("s")
    wid = s * NC + c

    row0 = s * ROWS_PER_TILE
    pltpu.sync_copy(zeros_hbm.at[pl.ds(0, ROWS_PER_TILE)],
                    acc_spmem.at[pl.ds(row0, ROWS_PER_TILE)])
    plsc.subcore_barrier()

    def step(g, carry):
        base = wid * EPW + g * CHUNK
        pltpu.sync_copy(src_hbm.at[pl.ds(base, CHUNK)], idx_src_v)
        pltpu.sync_copy(dst_hbm.at[pl.ds(base, CHUNK)], idx_dst_v)
        pltpu.async_copy(node_hbm.at[idx_src_v], node_rows_v, sem).wait()
        pltpu.sync_copy(edge_hbm.at[pl.ds(base, CHUNK)], edge_rows_v)
        pltpu.sync_copy(node_rows_v, acc_spmem.at[idx_dst_v], add=True)
        pltpu.sync_copy(edge_rows_v, acc_spmem.at[idx_dst_v], add=True)
        return carry

    lax.fori_loop(0, NCHUNK, step, 0)
    plsc.subcore_barrier()

    pltpu.sync_copy(acc_spmem.at[pl.ds(row0, ROWS_PER_TILE)],
                    out_hbm.at[c, pl.ds(row0, ROWS_PER_TILE)])


@jax.jit
def _sc_segment_sum(node_feat, edge_feat, src, dst):
    zeros = jnp.zeros((ROWS_PER_TILE, D), dtype=jnp.float32)
    mesh = plsc.VectorSubcoreMesh(core_axis_name="c", subcore_axis_name="s",
                                  num_cores=NC, num_subcores=NS)
    f = pl.kernel(
        _sc_body,
        out_type=jax.ShapeDtypeStruct((NC, N, D), jnp.float32),
        mesh=mesh,
        scratch_types=[
            pltpu.VMEM((CHUNK,), jnp.int32),
            pltpu.VMEM((CHUNK,), jnp.int32),
            pltpu.VMEM((CHUNK, D), jnp.float32),
            pltpu.VMEM((CHUNK, D), jnp.float32),
            pltpu.VMEM_SHARED((N, D), jnp.float32),
            pltpu.SemaphoreType.DMA,
        ],
    )
    return f(node_feat, edge_feat, src, dst, zeros)


def _mlp_body(x_ref, p_ref, W1_ref, b1_ref, gamma_ref, beta_ref, W2_ref,
              b2_ref, eps_ref, o_ref):
    neigh = p_ref[0] + p_ref[1]
    rst = (1.0 + eps_ref[0]) * x_ref[...] + neigh
    h = jnp.dot(rst, W1_ref[...], preferred_element_type=jnp.float32,
                precision=lax.Precision.HIGHEST) + b1_ref[...]
    mean = jnp.mean(h, axis=0)
    var = jnp.mean((h - mean) ** 2, axis=0)
    h = (h - mean) * (gamma_ref[...] * lax.rsqrt(var + 1e-5)) + beta_ref[...]
    h = jnp.maximum(h, 0.0)
    o_ref[...] = jnp.dot(h, W2_ref[...], preferred_element_type=jnp.float32,
                         precision=lax.Precision.HIGHEST) + b2_ref[...]


@jax.jit
def _tc_mlp(node_feat, partials, W1, b1, gamma, beta, W2, b2, eps):
    return pl.pallas_call(
        _mlp_body,
        out_shape=jax.ShapeDtypeStruct((N, D), jnp.float32),
        in_specs=[
            pl.BlockSpec(memory_space=pltpu.VMEM),
            pl.BlockSpec(memory_space=pltpu.VMEM),
            pl.BlockSpec(memory_space=pltpu.VMEM),
            pl.BlockSpec(memory_space=pltpu.VMEM),
            pl.BlockSpec(memory_space=pltpu.VMEM),
            pl.BlockSpec(memory_space=pltpu.VMEM),
            pl.BlockSpec(memory_space=pltpu.VMEM),
            pl.BlockSpec(memory_space=pltpu.VMEM),
            pl.BlockSpec(memory_space=pltpu.SMEM),
        ],
        out_specs=pl.BlockSpec(memory_space=pltpu.VMEM),
    )(node_feat, partials, W1, b1, gamma, beta, W2, b2, eps)


def kernel(node_feat, edge_feat, edge_index, W1, b1, gamma, beta, W2, b2, eps):
    src = edge_index[0]
    dst = edge_index[1]
    partials = _sc_segment_sum(node_feat, edge_feat, src, dst)
    return _tc_mlp(node_feat, partials, W1, b1, gamma, beta, W2, b2, eps)


# speedup vs baseline: 3.4150x; 3.4150x over previous
import jax
import jax.numpy as jnp
from jax import lax
from jax.experimental import pallas as pl
from jax.experimental.pallas import tpu as pltpu
from jax.experimental.pallas import tpu_sc as plsc

N = 10000
E = 320000
D = 128

NC = 2
NS = 16
NW = NC * NS
EPW = E // NW
CHUNK = 80
NCHUNK = EPW // CHUNK
N_PAD = 10240
ROWS_PER_TILE = N_PAD // NS


def _sc_body(node_hbm, edge_hbm, src_hbm, dst_hbm, zeros_hbm, out_hbm,
             idx_src_v, idx_dst_v, node_rows_v, edge_rows_v, acc_spmem, sem):
    c = lax.axis_index("c")
    s = lax.axis_index("s")
    wid = s * NC + c

    row0 = s * ROWS_PER_TILE
    pltpu.sync_copy(zeros_hbm.at[pl.ds(0, ROWS_PER_TILE)],
                    acc_spmem.at[pl.ds(row0, ROWS_PER_TILE)])
    plsc.subcore_barrier()

    def step(g, carry):
        base = wid * EPW + g * CHUNK
        pltpu.sync_copy(src_hbm.at[pl.ds(base, CHUNK)], idx_src_v)
        pltpu.sync_copy(dst_hbm.at[pl.ds(base, CHUNK)], idx_dst_v)
        pltpu.async_copy(node_hbm.at[idx_src_v], node_rows_v, sem).wait()
        pltpu.sync_copy(edge_hbm.at[pl.ds(base, CHUNK)], edge_rows_v)
        pltpu.sync_copy(node_rows_v, acc_spmem.at[idx_dst_v], add=True)
        pltpu.sync_copy(edge_rows_v, acc_spmem.at[idx_dst_v], add=True)
        return carry

    lax.fori_loop(0, NCHUNK, step, 0)
    plsc.subcore_barrier()

    pltpu.sync_copy(acc_spmem.at[pl.ds(row0, ROWS_PER_TILE)],
                    out_hbm.at[c, pl.ds(row0, ROWS_PER_TILE)])


@jax.jit
def _sc_segment_sum(node_feat, edge_feat, src, dst):
    zeros = jnp.zeros((ROWS_PER_TILE, D), dtype=jnp.float32)
    mesh = plsc.VectorSubcoreMesh(core_axis_name="c", subcore_axis_name="s",
                                  num_cores=NC, num_subcores=NS)
    f = pl.kernel(
        _sc_body,
        out_type=jax.ShapeDtypeStruct((NC, N_PAD, D), jnp.float32),
        mesh=mesh,
        scratch_types=[
            pltpu.VMEM((CHUNK,), jnp.int32),
            pltpu.VMEM((CHUNK,), jnp.int32),
            pltpu.VMEM((CHUNK, D), jnp.float32),
            pltpu.VMEM((CHUNK, D), jnp.float32),
            pltpu.VMEM_SHARED((N_PAD, D), jnp.float32),
            pltpu.SemaphoreType.DMA,
        ],
    )
    return f(node_feat, edge_feat, src, dst, zeros)


def _mlp_body(x_ref, p_ref, W1_ref, b1_ref, gamma_ref, beta_ref, W2_ref,
              b2_ref, eps_ref, o_ref):
    neigh = p_ref[0, :N] + p_ref[1, :N]
    rst = (1.0 + eps_ref[0]) * x_ref[...] + neigh
    h = jnp.dot(rst, W1_ref[...], preferred_element_type=jnp.float32,
                precision=lax.Precision.HIGHEST) + b1_ref[...]
    mean = jnp.mean(h, axis=0)
    var = jnp.mean((h - mean) ** 2, axis=0)
    h = (h - mean) * (gamma_ref[...] * lax.rsqrt(var + 1e-5)) + beta_ref[...]
    h = jnp.maximum(h, 0.0)
    o_ref[...] = jnp.dot(h, W2_ref[...], preferred_element_type=jnp.float32,
                         precision=lax.Precision.HIGHEST) + b2_ref[...]


@jax.jit
def _tc_mlp(node_feat, partials, W1, b1, gamma, beta, W2, b2, eps):
    return pl.pallas_call(
        _mlp_body,
        out_shape=jax.ShapeDtypeStruct((N, D), jnp.float32),
        in_specs=[
            pl.BlockSpec(memory_space=pltpu.VMEM),
            pl.BlockSpec(memory_space=pltpu.VMEM),
            pl.BlockSpec(memory_space=pltpu.VMEM),
            pl.BlockSpec(memory_space=pltpu.VMEM),
            pl.BlockSpec(memory_space=pltpu.VMEM),
            pl.BlockSpec(memory_space=pltpu.VMEM),
            pl.BlockSpec(memory_space=pltpu.VMEM),
            pl.BlockSpec(memory_space=pltpu.VMEM),
            pl.BlockSpec(memory_space=pltpu.SMEM),
        ],
        out_specs=pl.BlockSpec(memory_space=pltpu.VMEM),
    )(node_feat, partials, W1, b1, gamma, beta, W2, b2, eps)


def kernel(node_feat, edge_feat, edge_index, W1, b1, gamma, beta, W2, b2, eps):
    src = edge_index[0]
    dst = edge_index[1]
    partials = _sc_segment_sum(node_feat, edge_feat, src, dst)
    return _tc_mlp(node_feat, partials, W1, b1, gamma, beta, W2, b2, eps)


# speedup vs baseline: 5.3388x; 1.5634x over previous
import jax
import jax.numpy as jnp
from jax import lax
from jax.experimental import pallas as pl
from jax.experimental.pallas import tpu as pltpu
from jax.experimental.pallas import tpu_sc as plsc

N = 10000
E = 320000
D = 128

NC = 2
NS = 16
NW = NC * NS
EPW = E // NW
CHUNK = 80
NCHUNK = EPW // CHUNK
N_PAD = 10240
ROWS_PER_TILE = N_PAD // NS


NB = 2
NBLK = (NCHUNK - 1) // NB


def _sc_body(node_hbm, edge_hbm, src_hbm, dst_hbm, zeros_hbm, out_hbm,
             idx_src_v, idx_dst_v, node_rows_v, edge_rows_v, acc_spmem,
             sem_load, sem_gath, sem_scat):
    c = lax.axis_index("c")
    s = lax.axis_index("s")
    wid = s * NC + c

    row0 = s * ROWS_PER_TILE
    pltpu.sync_copy(zeros_hbm.at[pl.ds(0, ROWS_PER_TILE)],
                    acc_spmem.at[pl.ds(row0, ROWS_PER_TILE)])
    plsc.subcore_barrier()

    def block(t, carry):
        base_blk = wid * EPW + t * (NB * CHUNK)
        for b in range(NB):
            off = base_blk + b * CHUNK

            @pl.when(t > 0)
            def _drain(b=b):
                pltpu.make_async_copy(
                    node_rows_v.at[b], acc_spmem.at[idx_dst_v.at[b]],
                    sem_scat.at[b]).wait()
                pltpu.make_async_copy(
                    edge_rows_v.at[b], acc_spmem.at[idx_dst_v.at[b]],
                    sem_scat.at[b]).wait()

            pltpu.async_copy(src_hbm.at[pl.ds(off, CHUNK)], idx_src_v.at[b],
                             sem_load.at[b])
            pltpu.async_copy(dst_hbm.at[pl.ds(off, CHUNK)], idx_dst_v.at[b],
                             sem_load.at[b])
            pltpu.async_copy(edge_hbm.at[pl.ds(off, CHUNK)], edge_rows_v.at[b],
                             sem_load.at[b])
        for b in range(NB):
            off = base_blk + b * CHUNK
            pltpu.make_async_copy(src_hbm.at[pl.ds(off, CHUNK)],
                                  idx_src_v.at[b], sem_load.at[b]).wait()
            pltpu.make_async_copy(dst_hbm.at[pl.ds(off, CHUNK)],
                                  idx_dst_v.at[b], sem_load.at[b]).wait()
            pltpu.make_async_copy(edge_hbm.at[pl.ds(off, CHUNK)],
                                  edge_rows_v.at[b], sem_load.at[b]).wait()
            pltpu.async_copy(node_hbm.at[idx_src_v.at[b]], node_rows_v.at[b],
                             sem_gath.at[b])
        for b in range(NB):
            pltpu.make_async_copy(node_hbm.at[idx_src_v.at[b]],
                                  node_rows_v.at[b], sem_gath.at[b]).wait()
            pltpu.async_copy(node_rows_v.at[b], acc_spmem.at[idx_dst_v.at[b]],
                             sem_scat.at[b], add=True)
            pltpu.async_copy(edge_rows_v.at[b], acc_spmem.at[idx_dst_v.at[b]],
                             sem_scat.at[b], add=True)
        return carry

    lax.fori_loop(0, NBLK, block, 0)

    pltpu.make_async_copy(node_rows_v.at[0], acc_spmem.at[idx_dst_v.at[0]],
                          sem_scat.at[0]).wait()
    pltpu.make_async_copy(edge_rows_v.at[0], acc_spmem.at[idx_dst_v.at[0]],
                          sem_scat.at[0]).wait()
    off = wid * EPW + (NCHUNK - 1) * CHUNK
    pltpu.sync_copy(src_hbm.at[pl.ds(off, CHUNK)], idx_src_v.at[0])
    pltpu.sync_copy(dst_hbm.at[pl.ds(off, CHUNK)], idx_dst_v.at[0])
    pltpu.sync_copy(edge_hbm.at[pl.ds(off, CHUNK)], edge_rows_v.at[0])
    pltpu.async_copy(node_hbm.at[idx_src_v.at[0]], node_rows_v.at[0],
                     sem_gath.at[0]).wait()
    pltpu.sync_copy(node_rows_v.at[0], acc_spmem.at[idx_dst_v.at[0]], add=True)
    pltpu.sync_copy(edge_rows_v.at[0], acc_spmem.at[idx_dst_v.at[0]], add=True)
    pltpu.make_async_copy(node_rows_v.at[1], acc_spmem.at[idx_dst_v.at[1]],
                          sem_scat.at[1]).wait()
    pltpu.make_async_copy(edge_rows_v.at[1], acc_spmem.at[idx_dst_v.at[1]],
                          sem_scat.at[1]).wait()
    plsc.subcore_barrier()

    pltpu.sync_copy(acc_spmem.at[pl.ds(row0, ROWS_PER_TILE)],
                    out_hbm.at[c, pl.ds(row0, ROWS_PER_TILE)])


@jax.jit
def _sc_segment_sum(node_feat, edge_feat, src, dst):
    zeros = jnp.zeros((ROWS_PER_TILE, D), dtype=jnp.float32)
    mesh = plsc.VectorSubcoreMesh(core_axis_name="c", subcore_axis_name="s",
                                  num_cores=NC, num_subcores=NS)
    f = pl.kernel(
        _sc_body,
        out_type=jax.ShapeDtypeStruct((NC, N_PAD, D), jnp.float32),
        mesh=mesh,
        scratch_types=[
            pltpu.VMEM((NB, CHUNK), jnp.int32),
            pltpu.VMEM((NB, CHUNK), jnp.int32),
            pltpu.VMEM((NB, CHUNK, D), jnp.float32),
            pltpu.VMEM((NB, CHUNK, D), jnp.float32),
            pltpu.VMEM_SHARED((N_PAD, D), jnp.float32),
            pltpu.SemaphoreType.DMA((NB,)),
            pltpu.SemaphoreType.DMA((NB,)),
            pltpu.SemaphoreType.DMA((NB,)),
        ],
    )
    return f(node_feat, edge_feat, src, dst, zeros)


def _mlp_body(x_ref, p_ref, W1_ref, b1_ref, gamma_ref, beta_ref, W2_ref,
              b2_ref, eps_ref, o_ref):
    neigh = p_ref[0, :N] + p_ref[1, :N]
    rst = (1.0 + eps_ref[0]) * x_ref[...] + neigh
    h = jnp.dot(rst, W1_ref[...], preferred_element_type=jnp.float32,
                precision=lax.Precision.HIGHEST) + b1_ref[...]
    mean = jnp.mean(h, axis=0)
    var = jnp.mean((h - mean) ** 2, axis=0)
    h = (h - mean) * (gamma_ref[...] * lax.rsqrt(var + 1e-5)) + beta_ref[...]
    h = jnp.maximum(h, 0.0)
    o_ref[...] = jnp.dot(h, W2_ref[...], preferred_element_type=jnp.float32,
                         precision=lax.Precision.HIGHEST) + b2_ref[...]


@jax.jit
def _tc_mlp(node_feat, partials, W1, b1, gamma, beta, W2, b2, eps):
    return pl.pallas_call(
        _mlp_body,
        out_shape=jax.ShapeDtypeStruct((N, D), jnp.float32),
        in_specs=[
            pl.BlockSpec(memory_space=pltpu.VMEM),
            pl.BlockSpec(memory_space=pltpu.VMEM),
            pl.BlockSpec(memory_space=pltpu.VMEM),
            pl.BlockSpec(memory_space=pltpu.VMEM),
            pl.BlockSpec(memory_space=pltpu.VMEM),
            pl.BlockSpec(memory_space=pltpu.VMEM),
            pl.BlockSpec(memory_space=pltpu.VMEM),
            pl.BlockSpec(memory_space=pltpu.VMEM),
            pl.BlockSpec(memory_space=pltpu.SMEM),
        ],
        out_specs=pl.BlockSpec(memory_space=pltpu.VMEM),
    )(node_feat, partials, W1, b1, gamma, beta, W2, b2, eps)


def kernel(node_feat, edge_feat, edge_index, W1, b1, gamma, beta, W2, b2, eps):
    src = edge_index[0]
    dst = edge_index[1]
    partials = _sc_segment_sum(node_feat, edge_feat, src, dst)
    return _tc_mlp(node_feat, partials, W1, b1, gamma, beta, W2, b2, eps)


# speedup vs baseline: 7.0139x; 1.3137x over previous
import jax
import jax.numpy as jnp
from jax import lax
from jax.experimental import pallas as pl
from jax.experimental.pallas import tpu as pltpu
from jax.experimental.pallas import tpu_sc as plsc

N = 10000
E = 320000
D = 128

NC = 2
NS = 16
NW = NC * NS
EPW = E // NW
CHUNK = 40
NCHUNK = EPW // CHUNK
N_PAD = 10240
ROWS_PER_TILE = N_PAD // NS
NB = 3


def _sc_body(node_hbm, edge_hbm, src_hbm, dst_hbm, zeros_hbm, out_hbm,
             idx_src_v, idx_dst_v, node_rows_v, edge_rows_v, acc_spmem,
             sem_load, sem_gath, sem_scat):
    c = lax.axis_index("c")
    s = lax.axis_index("s")
    wid = s * NC + c
    base_w = wid * EPW

    row0 = s * ROWS_PER_TILE
    pltpu.sync_copy(zeros_hbm.at[pl.ds(0, ROWS_PER_TILE)],
                    acc_spmem.at[pl.ds(row0, ROWS_PER_TILE)])
    plsc.subcore_barrier()

    def loads(g):
        b = g % NB
        off = base_w + g * CHUNK
        pltpu.async_copy(src_hbm.at[pl.ds(off, CHUNK)], idx_src_v.at[b],
                         sem_load.at[b])
        pltpu.async_copy(dst_hbm.at[pl.ds(off, CHUNK)], idx_dst_v.at[b],
                         sem_load.at[b])
        pltpu.async_copy(edge_hbm.at[pl.ds(off, CHUNK)], edge_rows_v.at[b],
                         sem_load.at[b])

    def drain_scat(b):
        pltpu.make_async_copy(node_rows_v.at[b], acc_spmem.at[idx_dst_v.at[b]],
                              sem_scat.at[b]).wait()
        pltpu.make_async_copy(edge_rows_v.at[b], acc_spmem.at[idx_dst_v.at[b]],
                              sem_scat.at[b]).wait()

    def step(g, carry):
        @pl.when(g + 1 < NCHUNK)
        def _():
            @pl.when(g >= 2)
            def _():
                drain_scat((g + 1) % NB)
            loads(g + 1)

        b = g % NB
        off = base_w + g * CHUNK
        pltpu.make_async_copy(src_hbm.at[pl.ds(off, CHUNK)],
                              idx_src_v.at[b], sem_load.at[b]).wait()
        pltpu.make_async_copy(dst_hbm.at[pl.ds(off, CHUNK)],
                              idx_dst_v.at[b], sem_load.at[b]).wait()
        pltpu.make_async_copy(edge_hbm.at[pl.ds(off, CHUNK)],
                              edge_rows_v.at[b], sem_load.at[b]).wait()
        pltpu.async_copy(edge_rows_v.at[b], acc_spmem.at[idx_dst_v.at[b]],
                         sem_scat.at[b], add=True)
        pltpu.async_copy(node_hbm.at[idx_src_v.at[b]], node_rows_v.at[b],
                         sem_gath.at[b])

        @pl.when(g >= 1)
        def _():
            b1 = (g - 1) % NB
            pltpu.make_async_copy(node_hbm.at[idx_src_v.at[b1]],
                                  node_rows_v.at[b1], sem_gath.at[b1]).wait()
            pltpu.async_copy(node_rows_v.at[b1],
                             acc_spmem.at[idx_dst_v.at[b1]],
                             sem_scat.at[b1], add=True)
        return carry

    loads(0)
    lax.fori_loop(0, NCHUNK, step, 0)
    bl = (NCHUNK - 1) % NB
    pltpu.make_async_copy(node_hbm.at[idx_src_v.at[bl]],
                          node_rows_v.at[bl], sem_gath.at[bl]).wait()
    pltpu.async_copy(node_rows_v.at[bl], acc_spmem.at[idx_dst_v.at[bl]],
                     sem_scat.at[bl], add=True)
    drain_scat((NCHUNK - 3) % NB)
    drain_scat((NCHUNK - 2) % NB)
    drain_scat((NCHUNK - 1) % NB)
    plsc.subcore_barrier()

    pltpu.sync_copy(acc_spmem.at[pl.ds(row0, ROWS_PER_TILE)],
                    out_hbm.at[c, pl.ds(row0, ROWS_PER_TILE)])


@jax.jit
def _sc_segment_sum(node_feat, edge_feat, src, dst):
    zeros = jnp.zeros((ROWS_PER_TILE, D), dtype=jnp.float32)
    mesh = plsc.VectorSubcoreMesh(core_axis_name="c", subcore_axis_name="s",
                                  num_cores=NC, num_subcores=NS)
    f = pl.kernel(
        _sc_body,
        out_type=jax.ShapeDtypeStruct((NC, N_PAD, D), jnp.float32),
        mesh=mesh,
        scratch_types=[
            pltpu.VMEM((NB, CHUNK), jnp.int32),
            pltpu.VMEM((NB, CHUNK), jnp.int32),
            pltpu.VMEM((NB, CHUNK, D), jnp.float32),
            pltpu.VMEM((NB, CHUNK, D), jnp.float32),
            pltpu.VMEM_SHARED((N_PAD, D), jnp.float32),
            pltpu.SemaphoreType.DMA((NB,)),
            pltpu.SemaphoreType.DMA((NB,)),
            pltpu.SemaphoreType.DMA((NB,)),
        ],
    )
    return f(node_feat, edge_feat, src, dst, zeros)


def _mlp_body(x_ref, p_ref, W1_ref, b1_ref, gamma_ref, beta_ref, W2_ref,
              b2_ref, eps_ref, o_ref):
    neigh = p_ref[0, :N] + p_ref[1, :N]
    rst = (1.0 + eps_ref[0]) * x_ref[...] + neigh
    h = jnp.dot(rst, W1_ref[...], preferred_element_type=jnp.float32,
                precision=lax.Precision.HIGHEST) + b1_ref[...]
    mean = jnp.mean(h, axis=0)
    var = jnp.mean((h - mean) ** 2, axis=0)
    h = (h - mean) * (gamma_ref[...] * lax.rsqrt(var + 1e-5)) + beta_ref[...]
    h = jnp.maximum(h, 0.0)
    o_ref[...] = jnp.dot(h, W2_ref[...], preferred_element_type=jnp.float32,
                         precision=lax.Precision.HIGHEST) + b2_ref[...]


@jax.jit
def _tc_mlp(node_feat, partials, W1, b1, gamma, beta, W2, b2, eps):
    return pl.pallas_call(
        _mlp_body,
        out_shape=jax.ShapeDtypeStruct((N, D), jnp.float32),
        in_specs=[
            pl.BlockSpec(memory_space=pltpu.VMEM),
            pl.BlockSpec(memory_space=pltpu.VMEM),
            pl.BlockSpec(memory_space=pltpu.VMEM),
            pl.BlockSpec(memory_space=pltpu.VMEM),
            pl.BlockSpec(memory_space=pltpu.VMEM),
            pl.BlockSpec(memory_space=pltpu.VMEM),
            pl.BlockSpec(memory_space=pltpu.VMEM),
            pl.BlockSpec(memory_space=pltpu.VMEM),
            pl.BlockSpec(memory_space=pltpu.SMEM),
        ],
        out_specs=pl.BlockSpec(memory_space=pltpu.VMEM),
    )(node_feat, partials, W1, b1, gamma, beta, W2, b2, eps)


def kernel(node_feat, edge_feat, edge_index, W1, b1, gamma, beta, W2, b2, eps):
    src = edge_index[0]
    dst = edge_index[1]
    partials = _sc_segment_sum(node_feat, edge_feat, src, dst)
    return _tc_mlp(node_feat, partials, W1, b1, gamma, beta, W2, b2, eps)


# speedup vs baseline: 7.0932x; 1.0113x over previous
import jax
import jax.numpy as jnp
from jax import lax
from jax.experimental import pallas as pl
from jax.experimental.pallas import tpu as pltpu
from jax.experimental.pallas import tpu_sc as plsc

N = 10000
E = 320000
D = 128

NC = 2
NS = 16
NW = NC * NS
EPW = E // NW
CHUNK = 40
NCHUNK = EPW // CHUNK
N_PAD = 10240
ROWS_PER_TILE = N_PAD // NS
NB = 4


def _sc_body(node_hbm, edge_hbm, src_hbm, dst_hbm, zeros_hbm, out_hbm,
             idx_src_v, idx_dst_v, node_rows_v, edge_rows_v, acc_spmem,
             sem_load, sem_gath, sem_scat):
    c = lax.axis_index("c")
    s = lax.axis_index("s")
    wid = s * NC + c
    base_w = wid * EPW

    row0 = s * ROWS_PER_TILE
    pltpu.sync_copy(zeros_hbm.at[pl.ds(0, ROWS_PER_TILE)],
                    acc_spmem.at[pl.ds(row0, ROWS_PER_TILE)])
    plsc.subcore_barrier()

    def loads(g):
        b = g % NB
        off = base_w + g * CHUNK
        pltpu.async_copy(src_hbm.at[pl.ds(off, CHUNK)], idx_src_v.at[b],
                         sem_load.at[b])
        pltpu.async_copy(dst_hbm.at[pl.ds(off, CHUNK)], idx_dst_v.at[b],
                         sem_load.at[b])
        pltpu.async_copy(edge_hbm.at[pl.ds(off, CHUNK)], edge_rows_v.at[b],
                         sem_load.at[b])

    def drain_scat(b):
        pltpu.make_async_copy(node_rows_v.at[b], acc_spmem.at[idx_dst_v.at[b]],
                              sem_scat.at[b]).wait()
        pltpu.make_async_copy(edge_rows_v.at[b], acc_spmem.at[idx_dst_v.at[b]],
                              sem_scat.at[b]).wait()

    def step(g, carry):
        @pl.when(g + 1 < NCHUNK)
        def _():
            @pl.when(g >= 3)
            def _():
                drain_scat((g + 1) % NB)
            loads(g + 1)

        b = g % NB
        off = base_w + g * CHUNK
        pltpu.make_async_copy(src_hbm.at[pl.ds(off, CHUNK)],
                              idx_src_v.at[b], sem_load.at[b]).wait()
        pltpu.make_async_copy(dst_hbm.at[pl.ds(off, CHUNK)],
                              idx_dst_v.at[b], sem_load.at[b]).wait()
        pltpu.make_async_copy(edge_hbm.at[pl.ds(off, CHUNK)],
                              edge_rows_v.at[b], sem_load.at[b]).wait()
        pltpu.async_copy(edge_rows_v.at[b], acc_spmem.at[idx_dst_v.at[b]],
                         sem_scat.at[b], add=True)
        pltpu.async_copy(node_hbm.at[idx_src_v.at[b]], node_rows_v.at[b],
                         sem_gath.at[b])

        @pl.when(g >= 1)
        def _():
            b1 = (g - 1) % NB
            pltpu.make_async_copy(node_hbm.at[idx_src_v.at[b1]],
                                  node_rows_v.at[b1], sem_gath.at[b1]).wait()
            pltpu.async_copy(node_rows_v.at[b1],
                             acc_spmem.at[idx_dst_v.at[b1]],
                             sem_scat.at[b1], add=True)
        return carry

    loads(0)
    lax.fori_loop(0, NCHUNK, step, 0)
    bl = (NCHUNK - 1) % NB
    pltpu.make_async_copy(node_hbm.at[idx_src_v.at[bl]],
                          node_rows_v.at[bl], sem_gath.at[bl]).wait()
    pltpu.async_copy(node_rows_v.at[bl], acc_spmem.at[idx_dst_v.at[bl]],
                     sem_scat.at[bl], add=True)
    for k in range(NB, 0, -1):
        drain_scat((NCHUNK - k) % NB)
    plsc.subcore_barrier()

    pltpu.sync_copy(acc_spmem.at[pl.ds(row0, ROWS_PER_TILE)],
                    out_hbm.at[c, pl.ds(row0, ROWS_PER_TILE)])


@jax.jit
def _sc_segment_sum(node_feat, edge_feat, src, dst):
    zeros = jnp.zeros((ROWS_PER_TILE, D), dtype=jnp.float32)
    mesh = plsc.VectorSubcoreMesh(core_axis_name="c", subcore_axis_name="s",
                                  num_cores=NC, num_subcores=NS)
    f = pl.kernel(
        _sc_body,
        out_type=jax.ShapeDtypeStruct((NC, N_PAD, D), jnp.float32),
        mesh=mesh,
        scratch_types=[
            pltpu.VMEM((NB, CHUNK), jnp.int32),
            pltpu.VMEM((NB, CHUNK), jnp.int32),
            pltpu.VMEM((NB, CHUNK, D), jnp.float32),
            pltpu.VMEM((NB, CHUNK, D), jnp.float32),
            pltpu.VMEM_SHARED((N_PAD, D), jnp.float32),
            pltpu.SemaphoreType.DMA((NB,)),
            pltpu.SemaphoreType.DMA((NB,)),
            pltpu.SemaphoreType.DMA((NB,)),
        ],
    )
    return f(node_feat, edge_feat, src, dst, zeros)


def _mlp_body(x_ref, p_ref, W1_ref, b1_ref, gamma_ref, beta_ref, W2_ref,
              b2_ref, eps_ref, o_ref):
    neigh = p_ref[0, :N] + p_ref[1, :N]
    rst = (1.0 + eps_ref[0]) * x_ref[...] + neigh
    h = jnp.dot(rst, W1_ref[...], preferred_element_type=jnp.float32,
                precision=lax.Precision.HIGHEST) + b1_ref[...]
    mean = jnp.mean(h, axis=0)
    var = jnp.mean((h - mean) ** 2, axis=0)
    h = (h - mean) * (gamma_ref[...] * lax.rsqrt(var + 1e-5)) + beta_ref[...]
    h = jnp.maximum(h, 0.0)
    o_ref[...] = jnp.dot(h, W2_ref[...], preferred_element_type=jnp.float32,
                         precision=lax.Precision.HIGHEST) + b2_ref[...]


@jax.jit
def _tc_mlp(node_feat, partials, W1, b1, gamma, beta, W2, b2, eps):
    return pl.pallas_call(
        _mlp_body,
        out_shape=jax.ShapeDtypeStruct((N, D), jnp.float32),
        in_specs=[
            pl.BlockSpec(memory_space=pltpu.VMEM),
            pl.BlockSpec(memory_space=pltpu.VMEM),
            pl.BlockSpec(memory_space=pltpu.VMEM),
            pl.BlockSpec(memory_space=pltpu.VMEM),
            pl.BlockSpec(memory_space=pltpu.VMEM),
            pl.BlockSpec(memory_space=pltpu.VMEM),
            pl.BlockSpec(memory_space=pltpu.VMEM),
            pl.BlockSpec(memory_space=pltpu.VMEM),
            pl.BlockSpec(memory_space=pltpu.SMEM),
        ],
        out_specs=pl.BlockSpec(memory_space=pltpu.VMEM),
    )(node_feat, partials, W1, b1, gamma, beta, W2, b2, eps)


def kernel(node_feat, edge_feat, edge_index, W1, b1, gamma, beta, W2, b2, eps):
    src = edge_index[0]
    dst = edge_index[1]
    partials = _sc_segment_sum(node_feat, edge_feat, src, dst)
    return _tc_mlp(node_feat, partials, W1, b1, gamma, beta, W2, b2, eps)
